# Initial kernel scaffold; baseline (speedup 1.0000x reference)
import jax
import jax.numpy as jnp
from jax.experimental import pallas as pl


def kernel(x, edge_index, W1, b1, W2, b2, W3, b3):
    raise NotImplementedError("write your pallas kernel here")


# speedup vs baseline: 14.5583x; 14.5583x over previous
import functools

import jax
import jax.numpy as jnp
from jax import lax
from jax.experimental import pallas as pl
from jax.experimental.pallas import tpu as pltpu
from jax.experimental.pallas import tpu_sc as plsc

_f32 = jnp.float32
N = 10000
E = 320000
CH = 100
NR = E // CH
SB = 40
ROWS_PER_TILE = NR // 16
NQ = ROWS_PER_TILE // SB


def _mesh():
    return plsc.VectorSubcoreMesh(core_axis_name="c", subcore_axis_name="s")


@functools.partial(
    pl.kernel,
    out_type=jax.ShapeDtypeStruct((2 * N,), _f32),
    mesh=_mesh(),
    scratch_types=[
        pltpu.VMEM((32, CH), jnp.int32),
        pltpu.VMEM((112,), _f32),
        pltpu.VMEM((N,), _f32),
        pltpu.VMEM_SHARED((N,), _f32),
    ],
    name="gcn_deg",
)
def _deg_kernel(dst2_hbm, out_hbm, idx_d, ones_v, stage_v, acc_sh):
    c = lax.axis_index("c")
    s = lax.axis_index("s")
    for k in range(7):
        ones_v[pl.ds(k * 16, 16)] = jnp.full((16,), 1.0, _f32)

    @pl.when(s == 0)
    def _():
        def zf(j, carry):
            stage_v[pl.ds(j * 16, 16)] = jnp.zeros((16,), _f32)
            return carry
        lax.fori_loop(0, N // 16, zf, 0)
        pltpu.sync_copy(stage_v, acc_sh)

    plsc.subcore_barrier()
    w = c * 16 + s

    @pl.when(w < 25)
    def _():
        def outer(q, carry):
            pltpu.sync_copy(dst2_hbm.at[pl.ds(w * 128 + q * 32, 32)], idx_d)

            def body(g, icarry):
                pltpu.sync_copy(ones_v.at[pl.ds(0, CH)],
                                acc_sh.at[idx_d.at[g]], add=True)
                return icarry
            lax.fori_loop(0, 32, body, 0)
            return carry
        lax.fori_loop(0, 4, outer, 0)

    plsc.subcore_barrier()

    @pl.when(s < 5)
    def _():
        pltpu.sync_copy(acc_sh.at[pl.ds(s * 2000, 2000)],
                        stage_v.at[pl.ds(0, 2000)])
        pltpu.sync_copy(stage_v.at[pl.ds(0, 2000)],
                        out_hbm.at[pl.ds(c * N + s * 2000, 2000)])


def _zero_acc(stage_v, acc_sh, s):
    def zrow(r, carry):
        for k in range(128 // 16):
            stage_v[r, pl.ds(k * 16, 16)] = jnp.zeros((16,), _f32)
        return carry
    lax.fori_loop(0, 200, zrow, 0)

    @pl.when(s < 10)
    def _():
        for k in range(5):
            pltpu.sync_copy(stage_v, acc_sh.at[pl.ds(s * 1000 + k * 200, 200)])
    plsc.subcore_barrier()


def _write_out(stage_v, acc_sh, out_hbm, c, s):
    plsc.subcore_barrier()

    @pl.when(s < 10)
    def _():
        for k in range(5):
            sl = pl.ds(s * 1000 + k * 200, 200)
            pltpu.sync_copy(acc_sh.at[sl], stage_v)
            pltpu.sync_copy(stage_v, out_hbm.at[c, sl])


@functools.partial(
    pl.kernel,
    out_type=jax.ShapeDtypeStruct((2, N, 128), _f32),
    mesh=_mesh(),
    scratch_types=[
        pltpu.VMEM((SB, CH), jnp.int32),
        pltpu.VMEM((SB, CH), jnp.int32),
        pltpu.VMEM((CH, 128), _f32),
        pltpu.VMEM((200, 128), _f32),
        pltpu.VMEM_SHARED((N, 128), _f32),
    ],
    name="gcn_prop_split",
)
def _prop_split(hsl_hbm, hsr_hbm, src2_hbm, dst2_hbm, out_hbm,
                idx_s, idx_d, rows_v, stage_v, acc_sh):
    c = lax.axis_index("c")
    s = lax.axis_index("s")
    _zero_acc(stage_v, acc_sh, s)

    def edge_loop(hs_hbm):
        base_row = s * ROWS_PER_TILE

        def outer(q, carry):
            pltpu.sync_copy(src2_hbm.at[pl.ds(base_row + q * SB, SB)], idx_s)
            pltpu.sync_copy(dst2_hbm.at[pl.ds(base_row + q * SB, SB)], idx_d)

            def inner(g, icarry):
                pltpu.sync_copy(hs_hbm.at[idx_s.at[g]], rows_v)
                pltpu.sync_copy(rows_v, acc_sh.at[idx_d.at[g]], add=True)
                return icarry
            lax.fori_loop(0, SB, inner, 0)
            return carry
        lax.fori_loop(0, NQ, outer, 0)

    @pl.when(c == 0)
    def _():
        edge_loop(hsl_hbm)

    @pl.when(c == 1)
    def _():
        edge_loop(hsr_hbm)

    _write_out(stage_v, acc_sh, out_hbm, c, s)


@functools.partial(
    pl.kernel,
    out_type=jax.ShapeDtypeStruct((2, N, 128), _f32),
    mesh=_mesh(),
    scratch_types=[
        pltpu.VMEM((8, CH), jnp.int32),
        pltpu.VMEM((8, CH), jnp.int32),
        pltpu.VMEM((CH, 128), _f32),
        pltpu.VMEM((200, 128), _f32),
        pltpu.VMEM_SHARED((N, 128), _f32),
    ],
    name="gcn_prop_part",
)
def _prop_part(hs_hbm, src2_hbm, dst2_hbm, out_hbm,
               idx_s, idx_d, rows_v, stage_v, acc_sh):
    c = lax.axis_index("c")
    s = lax.axis_index("s")
    _zero_acc(stage_v, acc_sh, s)

    base_row = jnp.where(c == 0, s * 104, 1664 + s * 96)
    trips = jnp.where(c == 0, 13, 12)

    def outer(q, carry):
        pltpu.sync_copy(src2_hbm.at[pl.ds(base_row + q * 8, 8)], idx_s)
        pltpu.sync_copy(dst2_hbm.at[pl.ds(base_row + q * 8, 8)], idx_d)

        def inner(g, icarry):
            pltpu.sync_copy(hs_hbm.at[idx_s.at[g]], rows_v)
            pltpu.sync_copy(rows_v, acc_sh.at[idx_d.at[g]], add=True)
            return icarry
        lax.fori_loop(0, 8, inner, 0)
        return carry
    lax.fori_loop(0, trips, outer, 0)

    _write_out(stage_v, acc_sh, out_hbm, c, s)


_BM = 1000


def _pre_body(deg_ref, x_ref, w_ref, hsl_ref, hsr_ref, dinv_ref):
    deg = deg_ref[0] + deg_ref[1] + 1.0
    dinv = lax.rsqrt(deg)
    h = jnp.dot(x_ref[...], w_ref[...], preferred_element_type=_f32)
    hs = h * dinv
    hsl_ref[...] = hs[:, 0:128]
    hsr_ref[...] = hs[:, 128:256]
    dinv_ref[...] = dinv


def _pre(deg3, x, W1):
    return pl.pallas_call(
        _pre_body,
        grid=(N // _BM,),
        in_specs=[
            pl.BlockSpec((2, _BM, 1), lambda i: (0, i, 0)),
            pl.BlockSpec((_BM, 128), lambda i: (i, 0)),
            pl.BlockSpec((128, 256), lambda i: (0, 0)),
        ],
        out_specs=[
            pl.BlockSpec((_BM, 128), lambda i: (i, 0)),
            pl.BlockSpec((_BM, 128), lambda i: (i, 0)),
            pl.BlockSpec((_BM, 1), lambda i: (i, 0)),
        ],
        out_shape=[
            jax.ShapeDtypeStruct((N, 128), _f32),
            jax.ShapeDtypeStruct((N, 128), _f32),
            jax.ShapeDtypeStruct((N, 1), _f32),
        ],
    )(deg3, x, W1)


def _mid2_body(p_ref, hl_ref, hr_ref, dinv_ref, b_ref, w_ref, hs_ref):
    dinv = dinv_ref[...]
    a0 = jnp.maximum((p_ref[0] + hl_ref[...]) * dinv + b_ref[0:1, :], 0.0)
    a1 = jnp.maximum((p_ref[1] + hr_ref[...]) * dinv + b_ref[1:2, :], 0.0)
    h = (jnp.dot(a0, w_ref[0:128, :], preferred_element_type=_f32)
         + jnp.dot(a1, w_ref[128:, :], preferred_element_type=_f32))
    hs_ref[...] = h * dinv


def _mid2(p, hl, hr, dinv, b2, W):
    return pl.pallas_call(
        _mid2_body,
        grid=(N // _BM,),
        in_specs=[
            pl.BlockSpec((2, _BM, 128), lambda i: (0, i, 0)),
            pl.BlockSpec((_BM, 128), lambda i: (i, 0)),
            pl.BlockSpec((_BM, 128), lambda i: (i, 0)),
            pl.BlockSpec((_BM, 1), lambda i: (i, 0)),
            pl.BlockSpec((2, 128), lambda i: (0, 0)),
            pl.BlockSpec((256, 128), lambda i: (0, 0)),
        ],
        out_specs=pl.BlockSpec((_BM, 128), lambda i: (i, 0)),
        out_shape=jax.ShapeDtypeStruct((N, 128), _f32),
    )(p, hl, hr, dinv, b2, W)


def _mid3_body(p_ref, hs_prev_ref, dinv_ref, b_ref, w_ref, hs_ref):
    dinv = dinv_ref[...]
    p = p_ref[0] + p_ref[1] + hs_prev_ref[...]
    a = jnp.maximum(p * dinv + b_ref[...], 0.0)
    h = jnp.dot(a, w_ref[...], preferred_element_type=_f32)
    hs_ref[...] = h * dinv


def _mid3(p, hs_prev, dinv, b2, W):
    return pl.pallas_call(
        _mid3_body,
        grid=(N // _BM,),
        in_specs=[
            pl.BlockSpec((2, _BM, 128), lambda i: (0, i, 0)),
            pl.BlockSpec((_BM, 128), lambda i: (i, 0)),
            pl.BlockSpec((_BM, 1), lambda i: (i, 0)),
            pl.BlockSpec((1, 128), lambda i: (0, 0)),
            pl.BlockSpec((128, 128), lambda i: (0, 0)),
        ],
        out_specs=pl.BlockSpec((_BM, 128), lambda i: (i, 0)),
        out_shape=jax.ShapeDtypeStruct((N, 128), _f32),
    )(p, hs_prev, dinv, b2, W)


def _final_body(p_ref, hs_prev_ref, dinv_ref, b_ref, out_ref):
    p = p_ref[0] + p_ref[1] + hs_prev_ref[...]
    out_ref[...] = p * dinv_ref[...] + b_ref[...]


def _final(p, hs_prev, dinv, b2):
    return pl.pallas_call(
        _final_body,
        grid=(N // _BM,),
        in_specs=[
            pl.BlockSpec((2, _BM, 128), lambda i: (0, i, 0)),
            pl.BlockSpec((_BM, 128), lambda i: (i, 0)),
            pl.BlockSpec((_BM, 1), lambda i: (i, 0)),
            pl.BlockSpec((1, 128), lambda i: (0, 0)),
        ],
        out_specs=pl.BlockSpec((_BM, 128), lambda i: (i, 0)),
        out_shape=jax.ShapeDtypeStruct((N, 128), _f32),
    )(p, hs_prev, dinv, b2)


def kernel(x, edge_index, W1, b1, W2, b2, W3, b3):
    ei = edge_index.astype(jnp.int32)
    src2 = ei[0].reshape(NR, CH)
    dst2 = ei[1].reshape(NR, CH)

    deg = _deg_kernel(dst2)
    deg3 = deg.reshape(2, N, 1)
    hs1l, hs1r, dinv = _pre(deg3, x, W1)
    p1 = _prop_split(hs1l, hs1r, src2, dst2)
    hs2 = _mid2(p1, hs1l, hs1r, dinv, b1.reshape(2, 128), W2)
    p2 = _prop_part(hs2, src2, dst2)
    hs3 = _mid3(p2, hs2, dinv, b2.reshape(1, 128), W3)
    p3 = _prop_part(hs3, src2, dst2)
    return _final(p3, hs3, dinv, b3.reshape(1, 128))


# speedup vs baseline: 22.7195x; 1.5606x over previous
import functools

import jax
import jax.numpy as jnp
from jax import lax
from jax.experimental import pallas as pl
from jax.experimental.pallas import tpu as pltpu
from jax.experimental.pallas import tpu_sc as plsc

_f32 = jnp.float32
N = 10000
E = 320000
CH = 125
NR = E // CH
RPT_SPLIT = NR // 16
RPT_PART = NR // 32


def _mesh():
    return plsc.VectorSubcoreMesh(core_axis_name="c", subcore_axis_name="s")


@functools.partial(
    pl.kernel,
    out_type=jax.ShapeDtypeStruct((2 * N,), _f32),
    mesh=_mesh(),
    scratch_types=[
        pltpu.VMEM((RPT_PART, CH), jnp.int32),
        pltpu.VMEM((128,), _f32),
        pltpu.VMEM((N,), _f32),
        pltpu.VMEM_SHARED((N,), _f32),
    ],
    name="gcn_deg",
)
def _deg_kernel(dst2_hbm, out_hbm, idx_d, ones_v, stage_v, acc_sh):
    c = lax.axis_index("c")
    s = lax.axis_index("s")
    for k in range(8):
        ones_v[pl.ds(k * 16, 16)] = jnp.full((16,), 1.0, _f32)

    @pl.when(s == 0)
    def _():
        def zf(j, carry):
            stage_v[pl.ds(j * 16, 16)] = jnp.zeros((16,), _f32)
            return carry
        lax.fori_loop(0, N // 16, zf, 0)
        pltpu.sync_copy(stage_v, acc_sh)

    plsc.subcore_barrier()
    w = c * 16 + s
    pltpu.sync_copy(dst2_hbm.at[pl.ds(w * RPT_PART, RPT_PART)], idx_d)

    def body(g, carry):
        pltpu.sync_copy(ones_v.at[pl.ds(0, CH)],
                        acc_sh.at[idx_d.at[g]], add=True)
        return carry
    lax.fori_loop(0, RPT_PART, body, 0)
    plsc.subcore_barrier()

    @pl.when(s < 5)
    def _():
        pltpu.sync_copy(acc_sh.at[pl.ds(s * 2000, 2000)],
                        stage_v.at[pl.ds(0, 2000)])
        pltpu.sync_copy(stage_v.at[pl.ds(0, 2000)],
                        out_hbm.at[pl.ds(c * N + s * 2000, 2000)])


def _zero_acc(stage_v, acc_sh, s):
    def zrow(r, carry):
        for k in range(128 // 16):
            stage_v[r, pl.ds(k * 16, 16)] = jnp.zeros((16,), _f32)
        return carry
    lax.fori_loop(0, 40, zrow, 0)

    @pl.when(s < 10)
    def _():
        def cp(k, carry):
            pltpu.sync_copy(stage_v, acc_sh.at[pl.ds(s * 1000 + k * 40, 40)])
            return carry
        lax.fori_loop(0, 25, cp, 0)
    plsc.subcore_barrier()


def _write_out(stage_v, acc_sh, out_hbm, c, s):
    plsc.subcore_barrier()

    @pl.when(s < 10)
    def _():
        def cp(k, carry):
            sl = pl.ds(s * 1000 + k * 40, 40)
            pltpu.sync_copy(acc_sh.at[sl], stage_v)
            pltpu.sync_copy(stage_v, out_hbm.at[c, sl])
            return carry
        lax.fori_loop(0, 25, cp, 0)


_SB = 16


def _edge_pipeline(hs_hbm, src2_hbm, dst2_hbm, acc_sh, idx_s, idx_d,
                   rows0, rows1, sem0, sem1, base, n_blocks):
    def block(qb, carry):
        pltpu.sync_copy(src2_hbm.at[pl.ds(base + qb * _SB, _SB)], idx_s)
        pltpu.sync_copy(dst2_hbm.at[pl.ds(base + qb * _SB, _SB)], idx_d)
        pltpu.async_copy(hs_hbm.at[idx_s.at[0]], rows0, sem0)

        def outer(t, icarry):
            g0 = 2 * t
            pltpu.async_copy(hs_hbm.at[idx_s.at[g0 + 1]], rows1, sem1)
            pltpu.make_async_copy(hs_hbm.at[idx_s.at[g0]], rows0, sem0).wait()
            pltpu.sync_copy(rows0, acc_sh.at[idx_d.at[g0]], add=True)

            @pl.when(t < _SB // 2 - 1)
            def _():
                pltpu.async_copy(hs_hbm.at[idx_s.at[g0 + 2]], rows0, sem0)
            pltpu.make_async_copy(hs_hbm.at[idx_s.at[g0 + 1]], rows1,
                                  sem1).wait()
            pltpu.sync_copy(rows1, acc_sh.at[idx_d.at[g0 + 1]], add=True)
            return icarry
        lax.fori_loop(0, _SB // 2, outer, 0)
        return carry
    lax.fori_loop(0, n_blocks, block, 0)


@functools.partial(
    pl.kernel,
    out_type=jax.ShapeDtypeStruct((2, N, 128), _f32),
    mesh=_mesh(),
    scratch_types=[
        pltpu.VMEM((_SB, CH), jnp.int32),
        pltpu.VMEM((_SB, CH), jnp.int32),
        pltpu.VMEM((CH, 128), _f32),
        pltpu.VMEM((CH, 128), _f32),
        pltpu.VMEM((40, 128), _f32),
        pltpu.VMEM_SHARED((N, 128), _f32),
        pltpu.SemaphoreType.DMA,
        pltpu.SemaphoreType.DMA,
    ],
    name="gcn_prop_split",
)
def _prop_split(hsl_hbm, hsr_hbm, src2_hbm, dst2_hbm, out_hbm,
                idx_s, idx_d, rows0, rows1, stage_v, acc_sh, sem0, sem1):
    c = lax.axis_index("c")
    s = lax.axis_index("s")
    _zero_acc(stage_v, acc_sh, s)
    base = s * RPT_SPLIT

    @pl.when(c == 0)
    def _():
        _edge_pipeline(hsl_hbm, src2_hbm, dst2_hbm, acc_sh, idx_s, idx_d,
                       rows0, rows1, sem0, sem1, base, RPT_SPLIT // _SB)

    @pl.when(c == 1)
    def _():
        _edge_pipeline(hsr_hbm, src2_hbm, dst2_hbm, acc_sh, idx_s, idx_d,
                       rows0, rows1, sem0, sem1, base, RPT_SPLIT // _SB)

    _write_out(stage_v, acc_sh, out_hbm, c, s)


@functools.partial(
    pl.kernel,
    out_type=jax.ShapeDtypeStruct((2, N, 128), _f32),
    mesh=_mesh(),
    scratch_types=[
        pltpu.VMEM((_SB, CH), jnp.int32),
        pltpu.VMEM((_SB, CH), jnp.int32),
        pltpu.VMEM((CH, 128), _f32),
        pltpu.VMEM((CH, 128), _f32),
        pltpu.VMEM((40, 128), _f32),
        pltpu.VMEM_SHARED((N, 128), _f32),
        pltpu.SemaphoreType.DMA,
        pltpu.SemaphoreType.DMA,
    ],
    name="gcn_prop_part",
)
def _prop_part(hs_hbm, src2_hbm, dst2_hbm, out_hbm,
               idx_s, idx_d, rows0, rows1, stage_v, acc_sh, sem0, sem1):
    c = lax.axis_index("c")
    s = lax.axis_index("s")
    _zero_acc(stage_v, acc_sh, s)
    w = c * 16 + s
    _edge_pipeline(hs_hbm, src2_hbm, dst2_hbm, acc_sh, idx_s, idx_d,
                   rows0, rows1, sem0, sem1, w * RPT_PART, RPT_PART // _SB)
    _write_out(stage_v, acc_sh, out_hbm, c, s)


_BM = 1000


def _pre_body(deg_ref, x_ref, w_ref, hsl_ref, hsr_ref, dinv_ref):
    deg = deg_ref[0] + deg_ref[1] + 1.0
    dinv = lax.rsqrt(deg)
    h = jnp.dot(x_ref[...], w_ref[...], preferred_element_type=_f32)
    hs = h * dinv
    hsl_ref[...] = hs[:, 0:128]
    hsr_ref[...] = hs[:, 128:256]
    dinv_ref[...] = dinv


def _pre(deg3, x, W1):
    return pl.pallas_call(
        _pre_body,
        grid=(N // _BM,),
        in_specs=[
            pl.BlockSpec((2, _BM, 1), lambda i: (0, i, 0)),
            pl.BlockSpec((_BM, 128), lambda i: (i, 0)),
            pl.BlockSpec((128, 256), lambda i: (0, 0)),
        ],
        out_specs=[
            pl.BlockSpec((_BM, 128), lambda i: (i, 0)),
            pl.BlockSpec((_BM, 128), lambda i: (i, 0)),
            pl.BlockSpec((_BM, 1), lambda i: (i, 0)),
        ],
        out_shape=[
            jax.ShapeDtypeStruct((N, 128), _f32),
            jax.ShapeDtypeStruct((N, 128), _f32),
            jax.ShapeDtypeStruct((N, 1), _f32),
        ],
    )(deg3, x, W1)


def _mid2_body(p_ref, hl_ref, hr_ref, dinv_ref, b_ref, w_ref, hs_ref):
    dinv = dinv_ref[...]
    a0 = jnp.maximum((p_ref[0] + hl_ref[...]) * dinv + b_ref[0:1, :], 0.0)
    a1 = jnp.maximum((p_ref[1] + hr_ref[...]) * dinv + b_ref[1:2, :], 0.0)
    h = (jnp.dot(a0, w_ref[0:128, :], preferred_element_type=_f32)
         + jnp.dot(a1, w_ref[128:, :], preferred_element_type=_f32))
    hs_ref[...] = h * dinv


def _mid2(p, hl, hr, dinv, b2, W):
    return pl.pallas_call(
        _mid2_body,
        grid=(N // _BM,),
        in_specs=[
            pl.BlockSpec((2, _BM, 128), lambda i: (0, i, 0)),
            pl.BlockSpec((_BM, 128), lambda i: (i, 0)),
            pl.BlockSpec((_BM, 128), lambda i: (i, 0)),
            pl.BlockSpec((_BM, 1), lambda i: (i, 0)),
            pl.BlockSpec((2, 128), lambda i: (0, 0)),
            pl.BlockSpec((256, 128), lambda i: (0, 0)),
        ],
        out_specs=pl.BlockSpec((_BM, 128), lambda i: (i, 0)),
        out_shape=jax.ShapeDtypeStruct((N, 128), _f32),
    )(p, hl, hr, dinv, b2, W)


def _mid3_body(p_ref, hs_prev_ref, dinv_ref, b_ref, w_ref, hs_ref):
    dinv = dinv_ref[...]
    p = p_ref[0] + p_ref[1] + hs_prev_ref[...]
    a = jnp.maximum(p * dinv + b_ref[...], 0.0)
    h = jnp.dot(a, w_ref[...], preferred_element_type=_f32)
    hs_ref[...] = h * dinv


def _mid3(p, hs_prev, dinv, b2, W):
    return pl.pallas_call(
        _mid3_body,
        grid=(N // _BM,),
        in_specs=[
            pl.BlockSpec((2, _BM, 128), lambda i: (0, i, 0)),
            pl.BlockSpec((_BM, 128), lambda i: (i, 0)),
            pl.BlockSpec((_BM, 1), lambda i: (i, 0)),
            pl.BlockSpec((1, 128), lambda i: (0, 0)),
            pl.BlockSpec((128, 128), lambda i: (0, 0)),
        ],
        out_specs=pl.BlockSpec((_BM, 128), lambda i: (i, 0)),
        out_shape=jax.ShapeDtypeStruct((N, 128), _f32),
    )(p, hs_prev, dinv, b2, W)


def _final_body(p_ref, hs_prev_ref, dinv_ref, b_ref, out_ref):
    p = p_ref[0] + p_ref[1] + hs_prev_ref[...]
    out_ref[...] = p * dinv_ref[...] + b_ref[...]


def _final(p, hs_prev, dinv, b2):
    return pl.pallas_call(
        _final_body,
        grid=(N // _BM,),
        in_specs=[
            pl.BlockSpec((2, _BM, 128), lambda i: (0, i, 0)),
            pl.BlockSpec((_BM, 128), lambda i: (i, 0)),
            pl.BlockSpec((_BM, 1), lambda i: (i, 0)),
            pl.BlockSpec((1, 128), lambda i: (0, 0)),
        ],
        out_specs=pl.BlockSpec((_BM, 128), lambda i: (i, 0)),
        out_shape=jax.ShapeDtypeStruct((N, 128), _f32),
    )(p, hs_prev, dinv, b2)


def kernel(x, edge_index, W1, b1, W2, b2, W3, b3):
    ei = edge_index.astype(jnp.int32)
    src2 = ei[0].reshape(NR, CH)
    dst2 = ei[1].reshape(NR, CH)

    deg = _deg_kernel(dst2)
    deg3 = deg.reshape(2, N, 1)
    hs1l, hs1r, dinv = _pre(deg3, x, W1)
    p1 = _prop_split(hs1l, hs1r, src2, dst2)
    hs2 = _mid2(p1, hs1l, hs1r, dinv, b1.reshape(2, 128), W2)
    p2 = _prop_part(hs2, src2, dst2)
    hs3 = _mid3(p2, hs2, dinv, b2.reshape(1, 128), W3)
    p3 = _prop_part(hs3, src2, dst2)
    return _final(p3, hs3, dinv, b3.reshape(1, 128))


# speedup vs baseline: 26.9197x; 1.1849x over previous
import functools

import jax
import jax.numpy as jnp
from jax import lax
from jax.experimental import pallas as pl
from jax.experimental.pallas import tpu as pltpu
from jax.experimental.pallas import tpu_sc as plsc

_f32 = jnp.float32
N = 10000
E = 320000
CH = 125
NR = E // CH
RPT_SPLIT = NR // 16
RPT_PART = NR // 32


def _mesh():
    return plsc.VectorSubcoreMesh(core_axis_name="c", subcore_axis_name="s")


@functools.partial(
    pl.kernel,
    out_type=jax.ShapeDtypeStruct((2 * N,), _f32),
    mesh=_mesh(),
    scratch_types=[
        pltpu.VMEM((RPT_PART, CH), jnp.int32),
        pltpu.VMEM((128,), _f32),
        pltpu.VMEM((N,), _f32),
        pltpu.VMEM_SHARED((N,), _f32),
    ],
    name="gcn_deg",
)
def _deg_kernel(dst2_hbm, out_hbm, idx_d, ones_v, stage_v, acc_sh):
    c = lax.axis_index("c")
    s = lax.axis_index("s")
    for k in range(8):
        ones_v[pl.ds(k * 16, 16)] = jnp.full((16,), 1.0, _f32)

    @pl.when(s == 0)
    def _():
        def zf(j, carry):
            stage_v[pl.ds(j * 16, 16)] = jnp.zeros((16,), _f32)
            return carry
        lax.fori_loop(0, N // 16, zf, 0)
        pltpu.sync_copy(stage_v, acc_sh)

    plsc.subcore_barrier()
    w = c * 16 + s
    pltpu.sync_copy(dst2_hbm.at[pl.ds(w * RPT_PART, RPT_PART)], idx_d)

    def body(g, carry):
        pltpu.sync_copy(ones_v.at[pl.ds(0, CH)],
                        acc_sh.at[idx_d.at[g]], add=True)
        return carry
    lax.fori_loop(0, RPT_PART, body, 0)
    plsc.subcore_barrier()

    @pl.when(s < 5)
    def _():
        pltpu.sync_copy(acc_sh.at[pl.ds(s * 2000, 2000)],
                        stage_v.at[pl.ds(0, 2000)])
        pltpu.sync_copy(stage_v.at[pl.ds(0, 2000)],
                        out_hbm.at[pl.ds(c * N + s * 2000, 2000)])


def _zero_acc(stage_v, acc_sh, s):
    def zrow(r, carry):
        for k in range(128 // 16):
            stage_v[r, pl.ds(k * 16, 16)] = jnp.zeros((16,), _f32)
        return carry
    lax.fori_loop(0, 40, zrow, 0)

    for k in range(15):
        pltpu.sync_copy(stage_v, acc_sh.at[pl.ds(s * 625 + k * 40, 40)])
    pltpu.sync_copy(stage_v.at[pl.ds(0, 25), :],
                    acc_sh.at[pl.ds(s * 625 + 600, 25)])
    plsc.subcore_barrier()


def _write_out(stage_v, acc_sh, out_hbm, c, s):
    plsc.subcore_barrier()

    @pl.when(s < 10)
    def _():
        sl = pl.ds(s * 1000, 1000)
        pltpu.sync_copy(acc_sh.at[sl], out_hbm.at[c, sl])


_SB = 32


def _edge_pipeline(hs_hbm, src2_hbm, dst2_hbm, acc_sh, idx_s, idx_d,
                   rows0, rows1, sem0, sem1, base, n_blocks):
    def block(qb, carry):
        pltpu.sync_copy(src2_hbm.at[pl.ds(base + qb * _SB, _SB)], idx_s)
        pltpu.sync_copy(dst2_hbm.at[pl.ds(base + qb * _SB, _SB)], idx_d)
        pltpu.async_copy(hs_hbm.at[idx_s.at[0]], rows0, sem0)

        def outer(t, icarry):
            g0 = 2 * t
            pltpu.async_copy(hs_hbm.at[idx_s.at[g0 + 1]], rows1, sem1)
            pltpu.make_async_copy(hs_hbm.at[idx_s.at[g0]], rows0, sem0).wait()
            pltpu.sync_copy(rows0, acc_sh.at[idx_d.at[g0]], add=True)

            @pl.when(t < _SB // 2 - 1)
            def _():
                pltpu.async_copy(hs_hbm.at[idx_s.at[g0 + 2]], rows0, sem0)
            pltpu.make_async_copy(hs_hbm.at[idx_s.at[g0 + 1]], rows1,
                                  sem1).wait()
            pltpu.sync_copy(rows1, acc_sh.at[idx_d.at[g0 + 1]], add=True)
            return icarry
        lax.fori_loop(0, _SB // 2, outer, 0)
        return carry
    lax.fori_loop(0, n_blocks, block, 0)


@functools.partial(
    pl.kernel,
    out_type=jax.ShapeDtypeStruct((2, N, 128), _f32),
    mesh=_mesh(),
    scratch_types=[
        pltpu.VMEM((_SB, CH), jnp.int32),
        pltpu.VMEM((_SB, CH), jnp.int32),
        pltpu.VMEM((CH, 128), _f32),
        pltpu.VMEM((CH, 128), _f32),
        pltpu.VMEM((40, 128), _f32),
        pltpu.VMEM_SHARED((N, 128), _f32),
        pltpu.SemaphoreType.DMA,
        pltpu.SemaphoreType.DMA,
    ],
    name="gcn_prop_split",
)
def _prop_split(hsl_hbm, hsr_hbm, src2_hbm, dst2_hbm, out_hbm,
                idx_s, idx_d, rows0, rows1, stage_v, acc_sh, sem0, sem1):
    c = lax.axis_index("c")
    s = lax.axis_index("s")
    _zero_acc(stage_v, acc_sh, s)
    base = s * RPT_SPLIT

    @pl.when(c == 0)
    def _():
        _edge_pipeline(hsl_hbm, src2_hbm, dst2_hbm, acc_sh, idx_s, idx_d,
                       rows0, rows1, sem0, sem1, base, RPT_SPLIT // _SB)

    @pl.when(c == 1)
    def _():
        _edge_pipeline(hsr_hbm, src2_hbm, dst2_hbm, acc_sh, idx_s, idx_d,
                       rows0, rows1, sem0, sem1, base, RPT_SPLIT // _SB)

    _write_out(stage_v, acc_sh, out_hbm, c, s)


@functools.partial(
    pl.kernel,
    out_type=jax.ShapeDtypeStruct((2, N, 128), _f32),
    mesh=_mesh(),
    scratch_types=[
        pltpu.VMEM((_SB, CH), jnp.int32),
        pltpu.VMEM((_SB, CH), jnp.int32),
        pltpu.VMEM((CH, 128), _f32),
        pltpu.VMEM((CH, 128), _f32),
        pltpu.VMEM((40, 128), _f32),
        pltpu.VMEM_SHARED((N, 128), _f32),
        pltpu.SemaphoreType.DMA,
        pltpu.SemaphoreType.DMA,
    ],
    name="gcn_prop_part",
)
def _prop_part(hs_hbm, src2_hbm, dst2_hbm, out_hbm,
               idx_s, idx_d, rows0, rows1, stage_v, acc_sh, sem0, sem1):
    c = lax.axis_index("c")
    s = lax.axis_index("s")
    _zero_acc(stage_v, acc_sh, s)
    w = c * 16 + s
    _edge_pipeline(hs_hbm, src2_hbm, dst2_hbm, acc_sh, idx_s, idx_d,
                   rows0, rows1, sem0, sem1, w * RPT_PART, RPT_PART // _SB)
    _write_out(stage_v, acc_sh, out_hbm, c, s)


_BM = 1000


def _pre_body(deg_ref, x_ref, w_ref, hsl_ref, hsr_ref, dinv_ref):
    deg = deg_ref[0] + deg_ref[1] + 1.0
    dinv = lax.rsqrt(deg)
    h = jnp.dot(x_ref[...], w_ref[...], preferred_element_type=_f32)
    hs = h * dinv
    hsl_ref[...] = hs[:, 0:128]
    hsr_ref[...] = hs[:, 128:256]
    dinv_ref[...] = dinv


def _pre(deg3, x, W1):
    return pl.pallas_call(
        _pre_body,
        grid=(N // _BM,),
        in_specs=[
            pl.BlockSpec((2, _BM, 1), lambda i: (0, i, 0)),
            pl.BlockSpec((_BM, 128), lambda i: (i, 0)),
            pl.BlockSpec((128, 256), lambda i: (0, 0)),
        ],
        out_specs=[
            pl.BlockSpec((_BM, 128), lambda i: (i, 0)),
            pl.BlockSpec((_BM, 128), lambda i: (i, 0)),
            pl.BlockSpec((_BM, 1), lambda i: (i, 0)),
        ],
        out_shape=[
            jax.ShapeDtypeStruct((N, 128), _f32),
            jax.ShapeDtypeStruct((N, 128), _f32),
            jax.ShapeDtypeStruct((N, 1), _f32),
        ],
    )(deg3, x, W1)


def _mid2_body(p_ref, hl_ref, hr_ref, dinv_ref, b_ref, w_ref, hs_ref):
    dinv = dinv_ref[...]
    a0 = jnp.maximum((p_ref[0] + hl_ref[...]) * dinv + b_ref[0:1, :], 0.0)
    a1 = jnp.maximum((p_ref[1] + hr_ref[...]) * dinv + b_ref[1:2, :], 0.0)
    h = (jnp.dot(a0, w_ref[0:128, :], preferred_element_type=_f32)
         + jnp.dot(a1, w_ref[128:, :], preferred_element_type=_f32))
    hs_ref[...] = h * dinv


def _mid2(p, hl, hr, dinv, b2, W):
    return pl.pallas_call(
        _mid2_body,
        grid=(N // _BM,),
        in_specs=[
            pl.BlockSpec((2, _BM, 128), lambda i: (0, i, 0)),
            pl.BlockSpec((_BM, 128), lambda i: (i, 0)),
            pl.BlockSpec((_BM, 128), lambda i: (i, 0)),
            pl.BlockSpec((_BM, 1), lambda i: (i, 0)),
            pl.BlockSpec((2, 128), lambda i: (0, 0)),
            pl.BlockSpec((256, 128), lambda i: (0, 0)),
        ],
        out_specs=pl.BlockSpec((_BM, 128), lambda i: (i, 0)),
        out_shape=jax.ShapeDtypeStruct((N, 128), _f32),
    )(p, hl, hr, dinv, b2, W)


def _mid3_body(p_ref, hs_prev_ref, dinv_ref, b_ref, w_ref, hs_ref):
    dinv = dinv_ref[...]
    p = p_ref[0] + p_ref[1] + hs_prev_ref[...]
    a = jnp.maximum(p * dinv + b_ref[...], 0.0)
    h = jnp.dot(a, w_ref[...], preferred_element_type=_f32)
    hs_ref[...] = h * dinv


def _mid3(p, hs_prev, dinv, b2, W):
    return pl.pallas_call(
        _mid3_body,
        grid=(N // _BM,),
        in_specs=[
            pl.BlockSpec((2, _BM, 128), lambda i: (0, i, 0)),
            pl.BlockSpec((_BM, 128), lambda i: (i, 0)),
            pl.BlockSpec((_BM, 1), lambda i: (i, 0)),
            pl.BlockSpec((1, 128), lambda i: (0, 0)),
            pl.BlockSpec((128, 128), lambda i: (0, 0)),
        ],
        out_specs=pl.BlockSpec((_BM, 128), lambda i: (i, 0)),
        out_shape=jax.ShapeDtypeStruct((N, 128), _f32),
    )(p, hs_prev, dinv, b2, W)


def _final_body(p_ref, hs_prev_ref, dinv_ref, b_ref, out_ref):
    p = p_ref[0] + p_ref[1] + hs_prev_ref[...]
    out_ref[...] = p * dinv_ref[...] + b_ref[...]


def _final(p, hs_prev, dinv, b2):
    return pl.pallas_call(
        _final_body,
        grid=(N // _BM,),
        in_specs=[
            pl.BlockSpec((2, _BM, 128), lambda i: (0, i, 0)),
            pl.BlockSpec((_BM, 128), lambda i: (i, 0)),
            pl.BlockSpec((_BM, 1), lambda i: (i, 0)),
            pl.BlockSpec((1, 128), lambda i: (0, 0)),
        ],
        out_specs=pl.BlockSpec((_BM, 128), lambda i: (i, 0)),
        out_shape=jax.ShapeDtypeStruct((N, 128), _f32),
    )(p, hs_prev, dinv, b2)


def kernel(x, edge_index, W1, b1, W2, b2, W3, b3):
    ei = edge_index.astype(jnp.int32)
    src2 = ei[0].reshape(NR, CH)
    dst2 = ei[1].reshape(NR, CH)

    deg = _deg_kernel(dst2)
    deg3 = deg.reshape(2, N, 1)
    hs1l, hs1r, dinv = _pre(deg3, x, W1)
    p1 = _prop_split(hs1l, hs1r, src2, dst2)
    hs2 = _mid2(p1, hs1l, hs1r, dinv, b1.reshape(2, 128), W2)
    p2 = _prop_part(hs2, src2, dst2)
    hs3 = _mid3(p2, hs2, dinv, b2.reshape(1, 128), W3)
    p3 = _prop_part(hs3, src2, dst2)
    return _final(p3, hs3, dinv, b3.reshape(1, 128))
